# Initial kernel scaffold; baseline (speedup 1.0000x reference)
import jax
import jax.numpy as jnp
from jax.experimental import pallas as pl


def kernel(verts, faces):
    raise NotImplementedError("write your pallas kernel here")


# speedup vs baseline: 25.8210x; 25.8210x over previous
import functools

import jax
import jax.numpy as jnp
from jax import lax
from jax.experimental import pallas as pl
from jax.experimental.pallas import tpu as pltpu
from jax.experimental.pallas import tpu_sc as plsc

L = 16


def _rsqrt_nr(a):
    i = lax.bitcast_convert_type(a, jnp.int32)
    i = jnp.int32(0x5F3759DF) - (i >> 1)
    y = lax.bitcast_convert_type(i, jnp.float32)
    h = 0.5 * a
    for _ in range(3):
        y = y * (1.5 - h * y * y)
    return y


def _edge_len(dx, dy, dz):
    d2 = dx * dx + dy * dy + dz * dz
    return jnp.where(d2 > 0.0, d2 * _rsqrt_nr(d2), 0.0)


def _sc_body(nvp, fpt, nblk, fblk, fa, fb, fc, xh, yh, zh, zz, out1, out2,
             *refs):
    (xs, ys, zs, accv, acc2, dsta, dstb, dstc, cord, c2, stv, st1) = refs[:12]
    idxr = refs[12:24]
    valr = refs[24:36]
    gsem, ssem = refs[36:38]

    cid = lax.axis_index("c")
    sid = lax.axis_index("s")
    tid = cid * 16 + sid
    ch = nvp // 16
    ch4 = 4 * ch
    lo = sid * ch

    pltpu.sync_copy(xh.at[pl.ds(lo, ch)], st1)
    pltpu.sync_copy(st1, xs.at[pl.ds(lo, ch)])
    pltpu.sync_copy(yh.at[pl.ds(lo, ch)], st1)
    pltpu.sync_copy(st1, ys.at[pl.ds(lo, ch)])
    pltpu.sync_copy(zh.at[pl.ds(lo, ch)], st1)
    pltpu.sync_copy(st1, zs.at[pl.ds(lo, ch)])
    pltpu.sync_copy(zz.at[pl.ds(sid * ch4, ch4)], stv)
    pltpu.sync_copy(stv, accv.at[pl.ds(sid * ch4, ch4)])
    pltpu.sync_copy(zz.at[pl.ds(lo, ch)], st1)
    pltpu.sync_copy(st1, acc2.at[pl.ds(lo, ch)])

    two = jnp.full((L,), 2.0, jnp.float32)

    def fill_two(i, carry):
        c2[pl.ds(i * L, L)] = two
        return carry

    lax.fori_loop(0, fblk // L, fill_two, 0)

    plsc.subcore_barrier()

    lane = lax.iota(jnp.int32, L)

    for b in range(nblk):
        base = tid * fpt + b * fblk
        pltpu.sync_copy(fa.at[pl.ds(base, fblk)], dsta)
        pltpu.sync_copy(fb.at[pl.ds(base, fblk)], dstb)
        pltpu.sync_copy(fc.at[pl.ds(base, fblk)], dstc)
        descs = []
        for j, idx in enumerate((dsta, dstb, dstc)):
            for t, src in enumerate((xs, ys, zs)):
                descs.append(pltpu.async_copy(
                    src.at[idx], cord.at[pl.ds((3 * j + t) * fblk, fblk)],
                    gsem))
        for d in descs:
            d.wait()

        def compute(i, carry):
            off = i * L

            def cd(k):
                return cord[pl.ds(k * fblk + off, L)]

            ax, ay, az = cd(0), cd(1), cd(2)
            bx, by, bz = cd(3), cd(4), cd(5)
            cx, cy, cz = cd(6), cd(7), cd(8)
            l1 = _edge_len(bx - ax, by - ay, bz - az)
            l2 = _edge_len(cx - ax, cy - ay, cz - az)
            l3 = _edge_len(cx - bx, cy - by, cz - bz)
            vals = (
                (dsta, (bx + cx, by + cy, bz + cz, l1 + l2)),
                (dstb, (ax + cx, ay + cy, az + cz, l1 + l3)),
                (dstc, (ax + bx, ay + by, az + bz, l2 + l3)),
            )
            k = 0
            for dref, vs in vals:
                dst4 = dref[pl.ds(off, L)] << 2
                for col, v in enumerate(vs):
                    idxr[k][pl.ds(off, L)] = dst4 + col
                    valr[k][pl.ds(off, L)] = v
                    k += 1
            return carry

        lax.fori_loop(0, fblk // L, compute, 0)

        descs = []
        for k in range(12):
            descs.append(pltpu.async_copy(
                valr[k], accv.at[idxr[k]], ssem, add=True))
        for idx in (dsta, dstb, dstc):
            descs.append(pltpu.async_copy(
                c2, acc2.at[idx], ssem, add=True))
        for d in descs:
            d.wait()

    plsc.subcore_barrier()

    pltpu.sync_copy(accv.at[pl.ds(sid * ch4, ch4)], stv)
    pltpu.sync_copy(stv, out1.at[pl.ds(cid * 4 * nvp + sid * ch4, ch4)])
    pltpu.sync_copy(acc2.at[pl.ds(lo, ch)], st1)
    pltpu.sync_copy(st1, out2.at[pl.ds(cid * nvp + lo, ch)])


def _tc_body(nv, a1, a2, v, o):
    nvp = a2.shape[1]
    cnt = a2[0:1, :] + a2[1:2, :]
    tlx = a1[0, 0:1, :] + a1[1, 0:1, :]
    tly = a1[0, 1:2, :] + a1[1, 1:2, :]
    tlz = a1[0, 2:3, :] + a1[1, 2:3, :]
    te = a1[0, 3:4, :] + a1[1, 3:4, :]
    nx = v[0:1, :] * cnt - tlx
    ny = v[1:2, :] * cnt - tly
    nz = v[2:3, :] * cnt - tlz
    s = nx * nx + ny * ny + nz * nz
    r = jnp.sqrt(s) / te
    col = lax.broadcasted_iota(jnp.int32, (1, nvp), 1)
    r = jnp.where(col < nv, r, 0.0)
    o[:, :] = (jnp.sum(r) / nv).reshape(1, 1)


def kernel(verts, faces):
    nv = verts.shape[0]
    nf = faces.shape[0]
    fblk = 1600
    nw = 32
    nblk = -(-nf // (nw * fblk))
    nfp = nw * fblk * nblk
    fpt = nfp // nw
    nvp = -(-(nv + 1) // 128) * 128

    fpad = jnp.full((nfp - nf, 3), nv, jnp.int32)
    f_all = jnp.concatenate([faces, fpad], axis=0)
    fa, fb, fc = f_all[:, 0], f_all[:, 1], f_all[:, 2]
    vp = jnp.pad(verts, ((0, nvp - nv), (0, 0)))
    xh, yh, zh = vp[:, 0], vp[:, 1], vp[:, 2]
    zz = jnp.zeros((4 * nvp,), jnp.float32)

    mesh = plsc.VectorSubcoreMesh(core_axis_name="c", subcore_axis_name="s")
    scratch = [
        pltpu.VMEM_SHARED((nvp,), jnp.float32),
        pltpu.VMEM_SHARED((nvp,), jnp.float32),
        pltpu.VMEM_SHARED((nvp,), jnp.float32),
        pltpu.VMEM_SHARED((4 * nvp,), jnp.float32),
        pltpu.VMEM_SHARED((nvp,), jnp.float32),
        pltpu.VMEM((fblk,), jnp.int32),
        pltpu.VMEM((fblk,), jnp.int32),
        pltpu.VMEM((fblk,), jnp.int32),
        pltpu.VMEM((9 * fblk,), jnp.float32),
        pltpu.VMEM((fblk,), jnp.float32),
        pltpu.VMEM((4 * nvp // 16,), jnp.float32),
        pltpu.VMEM((nvp // 16,), jnp.float32),
    ]
    scratch += [pltpu.VMEM((fblk,), jnp.int32) for _ in range(12)]
    scratch += [pltpu.VMEM((fblk,), jnp.float32) for _ in range(12)]
    scratch += [pltpu.SemaphoreType.DMA, pltpu.SemaphoreType.DMA]
    sc = pl.kernel(
        functools.partial(_sc_body, nvp, fpt, nblk, fblk),
        out_type=(
            jax.ShapeDtypeStruct((2 * 4 * nvp,), jnp.float32),
            jax.ShapeDtypeStruct((2 * nvp,), jnp.float32),
        ),
        mesh=mesh,
        scratch_types=scratch,
    )
    p1, p2 = sc(fa, fb, fc, xh, yh, zh, zz)

    out = pl.pallas_call(
        functools.partial(_tc_body, nv),
        out_shape=jax.ShapeDtypeStruct((1, 1), jnp.float32),
    )(p1.reshape(2, nvp, 4).transpose(0, 2, 1), p2.reshape(2, nvp), vp.T)
    return out[0, 0]
